# baseline (speedup 1.0000x reference)
import functools

import jax
import jax.numpy as jnp
from jax import lax
from jax.experimental import pallas as pl
from jax.experimental.pallas import tpu as pltpu
from jax.experimental.pallas import tpu_sc as plsc

D_VOCAB = 100000
D_MODEL = 768
N_TOK = 8192
NUM_WORKERS = 32
ROWS_PER_WORKER = D_MODEL // NUM_WORKERS
LANES = 16


def _gather_body(x_hbm, w_hbm, outT_hbm, idx_v, row_v, val_v):
    c = lax.axis_index("c")
    s = lax.axis_index("s")
    wid = s * 2 + c

    pltpu.sync_copy(x_hbm, idx_v)

    def per_row(i, carry):
        d = wid * ROWS_PER_WORKER + i
        pltpu.sync_copy(w_hbm.at[d], row_v)

        def per_vec(j, carry2):
            iv = idx_v[pl.ds(j * LANES, LANES)]
            val_v[pl.ds(j * LANES, LANES)] = plsc.load_gather(row_v, [iv])
            return carry2

        lax.fori_loop(0, N_TOK // LANES, per_vec, 0, unroll=8)
        pltpu.sync_copy(val_v, outT_hbm.at[d])
        return carry

    lax.fori_loop(0, ROWS_PER_WORKER, per_row, 0)


@jax.jit
def _gather_rows(x_flat, w):
    mesh = plsc.VectorSubcoreMesh(core_axis_name="c", subcore_axis_name="s")
    fn = functools.partial(
        pl.kernel,
        out_type=jax.ShapeDtypeStruct((D_MODEL, N_TOK), jnp.float32),
        mesh=mesh,
        scratch_types=[
            pltpu.VMEM((N_TOK,), jnp.int32),
            pltpu.VMEM((D_VOCAB,), jnp.float32),
            pltpu.VMEM((N_TOK,), jnp.float32),
        ],
        compiler_params=pltpu.CompilerParams(needs_layout_passes=False),
    )(_gather_body)
    return fn(x_flat, w)


def kernel(x, W_E):
    b, s = x.shape
    x_flat = x.reshape(-1).astype(jnp.int32)
    outT = _gather_rows(x_flat, W_E)
    return jnp.transpose(outT).reshape(b, s, D_MODEL)
